# Initial kernel scaffold; baseline (speedup 1.0000x reference)
import jax
import jax.numpy as jnp
from jax.experimental import pallas as pl


def kernel(node_features, edge_index, edge_attr, enc_W, enc_b, eps, gin_W1_0, gin_b1_0, gin_W2_0, gin_b2_0, gin_W1_1, gin_b1_1, gin_W2_1, gin_b2_1, gin_W1_2, gin_b1_2, gin_W2_2, gin_b2_2, cls_W0, cls_b0, cls_g0, cls_be0, cls_W1, cls_b1, cls_g1, cls_be1, cls_W2, cls_b2):
    raise NotImplementedError("write your pallas kernel here")


# speedup vs baseline: 3.2782x; 3.2782x over previous
import jax
import jax.numpy as jnp
from jax import lax
from jax.experimental import pallas as pl
from jax.experimental.pallas import tpu as pltpu
from jax.experimental.pallas import tpu_sc as plsc

N = 10000
E = 320000
DF = 128
DE = 16
H = 64
HP = 128

NC = 2
NS = 16
NW = NC * NS

CH = 128
NROWS = 2560
EROWS = NROWS * CH
RPW = NROWS // NW
NSPLIT = 5
RPW2 = NROWS // NSPLIT // NW
PAD = 16
WT = 10
RPT = N // WT
SUB = 200

_f32 = jnp.float32
_PREC = lax.Precision.DEFAULT


def _sc_mesh():
    return plsc.VectorSubcoreMesh(
        core_axis_name="c", subcore_axis_name="s",
        num_cores=NC, num_subcores=NS)


def _segsum_body(x_hbm, pk_hbm, out0_hbm, out1_hbm,
                 pk_v, idxs_v, idxd_v, rows_v, buf_v, agg_sh, gsem):
    c = lax.axis_index("c")
    s = lax.axis_index("s")
    wid = s * NC + c

    def _zrow(r, carry):
        for j in range(HP // 16):
            buf_v[r, pl.ds(j * 16, 16)] = jnp.zeros((16,), _f32)
        return carry
    lax.fori_loop(0, SUB, _zrow, 0)

    @pl.when(s < WT)
    def _zero_main():
        def _zchunk(k, carry):
            pltpu.sync_copy(buf_v, agg_sh.at[pl.ds(s * RPT + k * SUB, SUB)])
            return carry
        lax.fori_loop(0, RPT // SUB, _zchunk, 0)

    @pl.when(s == WT)
    def _zero_pad():
        pltpu.sync_copy(buf_v.at[pl.ds(0, PAD)], agg_sh.at[pl.ds(N, PAD)])

    plsc.subcore_barrier()

    pltpu.sync_copy(pk_hbm.at[wid], pk_v)

    def _unpack(r, carry):
        for j in range(CH // 16):
            v = pk_v[r, pl.ds(j * 16, 16)]
            idxs_v[r, pl.ds(j * 16, 16)] = jnp.bitwise_and(v, 0xFFFF)
            idxd_v[r, pl.ds(j * 16, 16)] = lax.shift_right_logical(v, 16)
        return carry
    lax.fori_loop(0, RPW2, _unpack, 0)

    def _chunk(j, carry):
        pltpu.async_copy(x_hbm.at[idxs_v.at[j]], rows_v, gsem).wait()
        pltpu.sync_copy(rows_v, agg_sh.at[idxd_v.at[j]], add=True)
        return carry
    lax.fori_loop(0, RPW2, _chunk, 0)

    plsc.subcore_barrier()

    @pl.when(s < WT)
    def _writeout():
        def _wchunk(k, carry):
            r = s * RPT + k * SUB
            pltpu.sync_copy(agg_sh.at[pl.ds(r, SUB)], buf_v)

            @pl.when(c == 0)
            def _w0():
                pltpu.sync_copy(buf_v, out0_hbm.at[pl.ds(r, SUB)])

            @pl.when(c == 1)
            def _w1():
                pltpu.sync_copy(buf_v, out1_hbm.at[pl.ds(r, SUB)])
            return carry
        lax.fori_loop(0, RPT // SUB, _wchunk, 0)


def _segsum(x, pk2d):
    kern = pl.kernel(
        _segsum_body,
        out_type=(jax.ShapeDtypeStruct((N, HP), _f32),
                  jax.ShapeDtypeStruct((N, HP), _f32)),
        mesh=_sc_mesh(),
        scratch_types=[
            pltpu.VMEM((RPW2, CH), jnp.int32),
            pltpu.VMEM((RPW2, CH), jnp.int32),
            pltpu.VMEM((RPW2, CH), jnp.int32),
            pltpu.VMEM((CH, HP), _f32),
            pltpu.VMEM((SUB, HP), _f32),
            pltpu.VMEM_SHARED((N + PAD, HP), _f32),
            pltpu.SemaphoreType.DMA,
        ],
    )
    return kern(x, pk2d)


def _gather2_body(u_hbm, v_hbm, src_hbm, dst_hbm, gs_hbm, gd_hbm,
                  idxs_v, idxd_v, rs_v, rd_v, sem_s, sem_d):
    c = lax.axis_index("c")
    s = lax.axis_index("s")
    wid = s * NC + c
    row0 = wid * RPW
    pltpu.sync_copy(src_hbm.at[pl.ds(row0, RPW)], idxs_v)
    pltpu.sync_copy(dst_hbm.at[pl.ds(row0, RPW)], idxd_v)

    def _chunk(j, carry):
        g1 = pltpu.async_copy(u_hbm.at[idxs_v.at[j]], rs_v, sem_s)
        g2 = pltpu.async_copy(v_hbm.at[idxd_v.at[j]], rd_v, sem_d)
        g1.wait()
        g2.wait()
        base = (row0 + j) * CH
        pltpu.sync_copy(rs_v, gs_hbm.at[pl.ds(base, CH)])
        pltpu.sync_copy(rd_v, gd_hbm.at[pl.ds(base, CH)])
        return carry
    lax.fori_loop(0, RPW, _chunk, 0)


def _gather2(u, v, src2d, dst2d):
    kern = pl.kernel(
        _gather2_body,
        out_type=(jax.ShapeDtypeStruct((EROWS, HP), _f32),
                  jax.ShapeDtypeStruct((EROWS, HP), _f32)),
        mesh=_sc_mesh(),
        scratch_types=[
            pltpu.VMEM((RPW, CH), jnp.int32),
            pltpu.VMEM((RPW, CH), jnp.int32),
            pltpu.VMEM((CH, HP), _f32),
            pltpu.VMEM((CH, HP), _f32),
            pltpu.SemaphoreType.DMA,
            pltpu.SemaphoreType.DMA,
        ],
    )
    return kern(u, v, src2d, dst2d)


def _pad_cols(h):
    return jnp.concatenate([h, jnp.zeros_like(h)], axis=1)


def _mm_relu_body(x_ref, w_ref, b_ref, o_ref):
    h = jnp.maximum(
        jnp.dot(x_ref[...], w_ref[...], preferred_element_type=_f32, precision=_PREC)
        + b_ref[...], 0.0)
    o_ref[...] = _pad_cols(h)


def _encode(nf, W, b):
    RB = 2000
    return pl.pallas_call(
        _mm_relu_body,
        grid=(N // RB,),
        in_specs=[pl.BlockSpec((RB, DF), lambda i: (i, 0)),
                  pl.BlockSpec((DF, H), lambda i: (0, 0)),
                  pl.BlockSpec((1, H), lambda i: (0, 0))],
        out_specs=pl.BlockSpec((RB, HP), lambda i: (i, 0)),
        out_shape=jax.ShapeDtypeStruct((N, HP), _f32),
    )(nf, W, b.reshape(1, H))


def _gin_body(x_ref, *rest):
    (p_refs, (scl_ref, w1_ref, b1_ref, w2_ref, b2_ref, o_ref)) = (
        rest[:2 * NSPLIT], rest[2 * NSPLIT:])
    psum = p_refs[0][...]
    for p in p_refs[1:]:
        psum = psum + p[...]
    xin = x_ref[...][:, :H] * scl_ref[...] + psum[:, :H]
    h = jnp.maximum(
        jnp.dot(xin, w1_ref[...], preferred_element_type=_f32, precision=_PREC)
        + b1_ref[...], 0.0)
    o = jnp.maximum(
        jnp.dot(h, w2_ref[...], preferred_element_type=_f32, precision=_PREC)
        + b2_ref[...], 0.0)
    o_ref[...] = _pad_cols(o)


def _gin_mlp(x, parts, scl, W1, b1, W2, b2):
    RB = 2000
    return pl.pallas_call(
        _gin_body,
        grid=(N // RB,),
        in_specs=([pl.BlockSpec((RB, HP), lambda i: (i, 0))]
                  * (1 + 2 * NSPLIT)
                  + [pl.BlockSpec((1, H), lambda i: (0, 0)),
                     pl.BlockSpec((H, 2 * H), lambda i: (0, 0)),
                     pl.BlockSpec((1, 2 * H), lambda i: (0, 0)),
                     pl.BlockSpec((2 * H, H), lambda i: (0, 0)),
                     pl.BlockSpec((1, H), lambda i: (0, 0))]),
        out_specs=pl.BlockSpec((RB, HP), lambda i: (i, 0)),
        out_shape=jax.ShapeDtypeStruct((N, HP), _f32),
    )(x, *parts, scl, W1, b1.reshape(1, 2 * H), W2, b2.reshape(1, H))


def _uv_body(x_ref, ws_ref, wd_ref, u_ref, v_ref):
    xin = x_ref[...][:, :H]
    u_ref[...] = jnp.dot(xin, ws_ref[...], preferred_element_type=_f32, precision=_PREC)
    v_ref[...] = jnp.dot(xin, wd_ref[...], preferred_element_type=_f32, precision=_PREC)


def _uv(x, W0s, W0d):
    RB = 2000
    return pl.pallas_call(
        _uv_body,
        grid=(N // RB,),
        in_specs=[pl.BlockSpec((RB, HP), lambda i: (i, 0)),
                  pl.BlockSpec((H, HP), lambda i: (0, 0)),
                  pl.BlockSpec((H, HP), lambda i: (0, 0))],
        out_specs=[pl.BlockSpec((RB, HP), lambda i: (i, 0)),
                   pl.BlockSpec((RB, HP), lambda i: (i, 0))],
        out_shape=[jax.ShapeDtypeStruct((N, HP), _f32),
                   jax.ShapeDtypeStruct((N, HP), _f32)],
    )(x, W0s, W0d)


EB = 8000
EG = E // EB


def _clsA_body(gs_ref, gd_ref, ea_ref, we_ref, b_ref,
               h_ref, su_ref, sq_ref):
    h = (gs_ref[...] + gd_ref[...]
         + jnp.dot(ea_ref[...], we_ref[...], preferred_element_type=_f32, precision=_PREC)
         + b_ref[...])
    h_ref[...] = h

    @pl.when(pl.program_id(0) == 0)
    def _init():
        su_ref[...] = jnp.zeros_like(su_ref)
        sq_ref[...] = jnp.zeros_like(sq_ref)

    su_ref[...] = su_ref[...] + jnp.sum(h, axis=0, keepdims=True)
    sq_ref[...] = sq_ref[...] + jnp.sum(h * h, axis=0, keepdims=True)


def _clsA(gs, gd, ea, W0e, b0):
    D0 = 128
    return pl.pallas_call(
        _clsA_body,
        grid=(EG,),
        in_specs=[pl.BlockSpec((EB, HP), lambda i: (i, 0)),
                  pl.BlockSpec((EB, HP), lambda i: (i, 0)),
                  pl.BlockSpec((EB, DE), lambda i: (i, 0)),
                  pl.BlockSpec((DE, D0), lambda i: (0, 0)),
                  pl.BlockSpec((1, D0), lambda i: (0, 0))],
        out_specs=[pl.BlockSpec((EB, D0), lambda i: (i, 0)),
                   pl.BlockSpec((1, D0), lambda i: (0, 0)),
                   pl.BlockSpec((1, D0), lambda i: (0, 0))],
        out_shape=[jax.ShapeDtypeStruct((E, D0), _f32),
                   jax.ShapeDtypeStruct((1, D0), _f32),
                   jax.ShapeDtypeStruct((1, D0), _f32)],
    )(gs, gd, ea, W0e, b0.reshape(1, D0))


def _bn_from_stats(su, sq, g, be):
    mu = su * (1.0 / E)
    var = sq * (1.0 / E) - mu * mu
    sc = g * lax.rsqrt(var + 1e-5)
    return sc, be - mu * sc


def _clsB_body(h1_ref, su_ref, sq_ref, g_ref, be_ref, w_ref, b_ref,
               h2_ref, su2_ref, sq2_ref):
    sc, sh = _bn_from_stats(su_ref[...], sq_ref[...], g_ref[...], be_ref[...])
    a = jnp.maximum(h1_ref[...] * sc + sh, 0.0)
    h2 = jnp.dot(a, w_ref[...], preferred_element_type=_f32, precision=_PREC) + b_ref[...]
    h2_ref[...] = h2

    @pl.when(pl.program_id(0) == 0)
    def _init():
        su2_ref[...] = jnp.zeros_like(su2_ref)
        sq2_ref[...] = jnp.zeros_like(sq2_ref)

    su2_ref[...] = su2_ref[...] + jnp.sum(h2, axis=0, keepdims=True)
    sq2_ref[...] = sq2_ref[...] + jnp.sum(h2 * h2, axis=0, keepdims=True)


def _clsB(h1, su1, sq1, g0, be0, W1, b1):
    D0, D1 = 128, 64
    return pl.pallas_call(
        _clsB_body,
        grid=(EG,),
        in_specs=[pl.BlockSpec((EB, D0), lambda i: (i, 0)),
                  pl.BlockSpec((1, D0), lambda i: (0, 0)),
                  pl.BlockSpec((1, D0), lambda i: (0, 0)),
                  pl.BlockSpec((1, D0), lambda i: (0, 0)),
                  pl.BlockSpec((1, D0), lambda i: (0, 0)),
                  pl.BlockSpec((D0, D1), lambda i: (0, 0)),
                  pl.BlockSpec((1, D1), lambda i: (0, 0))],
        out_specs=[pl.BlockSpec((EB, D1), lambda i: (i, 0)),
                   pl.BlockSpec((1, D1), lambda i: (0, 0)),
                   pl.BlockSpec((1, D1), lambda i: (0, 0))],
        out_shape=[jax.ShapeDtypeStruct((E, D1), _f32),
                   jax.ShapeDtypeStruct((1, D1), _f32),
                   jax.ShapeDtypeStruct((1, D1), _f32)],
    )(h1, su1, sq1, g0.reshape(1, D0), be0.reshape(1, D0), W1,
      b1.reshape(1, D1))


def _clsC_body(h2_ref, su_ref, sq_ref, g_ref, be_ref, w_ref, b_ref, o_ref):
    sc, sh = _bn_from_stats(su_ref[...], sq_ref[...], g_ref[...], be_ref[...])
    a = jnp.maximum(h2_ref[...] * sc + sh, 0.0)
    o_ref[...] = (jnp.dot(a, w_ref[...], preferred_element_type=_f32, precision=_PREC)
                  + b_ref[...])


def _clsC(h2, su2, sq2, g1, be1, W2, b2):
    D1, D2 = 64, 2
    return pl.pallas_call(
        _clsC_body,
        grid=(EG,),
        in_specs=[pl.BlockSpec((EB, D1), lambda i: (i, 0)),
                  pl.BlockSpec((1, D1), lambda i: (0, 0)),
                  pl.BlockSpec((1, D1), lambda i: (0, 0)),
                  pl.BlockSpec((1, D1), lambda i: (0, 0)),
                  pl.BlockSpec((1, D1), lambda i: (0, 0)),
                  pl.BlockSpec((D1, D2), lambda i: (0, 0)),
                  pl.BlockSpec((1, D2), lambda i: (0, 0))],
        out_specs=pl.BlockSpec((EB, D2), lambda i: (i, 0)),
        out_shape=jax.ShapeDtypeStruct((E, D2), _f32),
    )(h2, su2, sq2, g1.reshape(1, D1), be1.reshape(1, D1), W2,
      b2.reshape(1, D2))


def kernel(node_features, edge_index, edge_attr, enc_W, enc_b, eps,
           gin_W1_0, gin_b1_0, gin_W2_0, gin_b2_0,
           gin_W1_1, gin_b1_1, gin_W2_1, gin_b2_1,
           gin_W1_2, gin_b1_2, gin_W2_2, gin_b2_2,
           cls_W0, cls_b0, cls_g0, cls_be0,
           cls_W1, cls_b1, cls_g1, cls_be1,
           cls_W2, cls_b2):
    src = edge_index[0].astype(jnp.int32)
    dst = edge_index[1].astype(jnp.int32)
    npad = EROWS - E
    padi = jnp.arange(npad, dtype=jnp.int32) & (PAD - 1)
    src2d = jnp.concatenate([src, padi]).reshape(NROWS, CH)
    dst2d_g = jnp.concatenate([dst, padi]).reshape(NROWS, CH)
    pk2d = src2d | ((jnp.concatenate([dst, N + padi])
                     .reshape(NROWS, CH)) << 16)

    x = _encode(node_features, enc_W, enc_b)
    gin = ((gin_W1_0, gin_b1_0, gin_W2_0, gin_b2_0),
           (gin_W1_1, gin_b1_1, gin_W2_1, gin_b2_1),
           (gin_W1_2, gin_b1_2, gin_W2_2, gin_b2_2))
    pk4d = pk2d.reshape(NSPLIT, NW, RPW2, CH)
    for i, (W1, b1, W2, b2) in enumerate(gin):
        parts = []
        for k in range(NSPLIT):
            parts.extend(_segsum(x, pk4d[k]))
        scl = jnp.full((1, H), 1.0 + eps[i], _f32)
        x = _gin_mlp(x, parts, scl, W1, b1, W2, b2)

    u, v = _uv(x, cls_W0[:H], cls_W0[H:2 * H])
    gs, gd = _gather2(u, v, src2d, dst2d_g)
    h1, su1, sq1 = _clsA(gs, gd, edge_attr, cls_W0[2 * H:], cls_b0)
    h2, su2, sq2 = _clsB(h1, su1, sq1, cls_g0, cls_be0, cls_W1, cls_b1)
    return _clsC(h2, su2, sq2, cls_g1, cls_be1, cls_W2, cls_b2)
